# Initial kernel scaffold; baseline (speedup 1.0000x reference)
import functools

import jax
import jax.numpy as jnp
from jax import lax
from jax.experimental import pallas as pl
from jax.experimental.pallas import tpu as pltpu
from jax.experimental.pallas import tpu_sc as plsc

NC = 2
NS = 16
NW = NC * NS
CH = 128


def _sc_aggregate(x, srcm, dstm, zeros, n_acc, k, rows_per_sub):
    d = x.shape[1]
    wchunks = rows_per_sub // CH
    mesh = plsc.VectorSubcoreMesh(core_axis_name="c", subcore_axis_name="s")

    @functools.partial(
        pl.kernel,
        out_type=jax.ShapeDtypeStruct((NC, n_acc, d), jnp.float32),
        mesh=mesh,
        scratch_types=[
            pltpu.VMEM((k, CH), jnp.int32),
            pltpu.VMEM((k, CH), jnp.int32),
            pltpu.VMEM((CH, d), jnp.float32),
            pltpu.VMEM_SHARED((n_acc, d), jnp.float32),
            pltpu.SemaphoreType.DMA,
        ],
    )
    def body(x_hbm, src_hbm, dst_hbm, zero_hbm, p_hbm, src_v, dst_v, rows_v,
             acc, sem):
        c = lax.axis_index("c")
        s = lax.axis_index("s")
        wid = s * NC + c

        pltpu.sync_copy(zero_hbm, rows_v)
        for t in range(wchunks):
            pltpu.sync_copy(
                rows_v, acc.at[pl.ds(s * rows_per_sub + t * CH, CH)])
        plsc.subcore_barrier()

        pltpu.sync_copy(src_hbm.at[pl.ds(wid * k, k)], src_v)
        pltpu.sync_copy(dst_hbm.at[pl.ds(wid * k, k)], dst_v)

        def step(j, carry):
            pltpu.async_copy(x_hbm.at[src_v.at[j]], rows_v, sem).wait()
            pltpu.sync_copy(rows_v, acc.at[dst_v.at[j]], add=True)
            return carry

        lax.fori_loop(0, k, step, 0)
        plsc.subcore_barrier()

        def wstep(t, carry):
            r0 = s * rows_per_sub + t * CH
            pltpu.sync_copy(acc.at[pl.ds(r0, CH)], rows_v)
            pltpu.sync_copy(rows_v, p_hbm.at[c, pl.ds(r0, CH)])
            return carry

        lax.fori_loop(0, wchunks, wstep, 0)

    return body(x, srcm, dstm, zeros)


def _tc_combine_matmul(p, w, n_out, block):
    d_in, d_out = w.shape

    def body(p_ref, w_ref, o_ref):
        a = p_ref[0] + p_ref[1]
        o_ref[...] = jnp.dot(a, w_ref[...],
                             preferred_element_type=jnp.float32)

    return pl.pallas_call(
        body,
        grid=(n_out // block,),
        in_specs=[
            pl.BlockSpec((NC, block, d_in), lambda i: (0, i, 0)),
            pl.BlockSpec((d_in, d_out), lambda i: (0, 0)),
        ],
        out_specs=pl.BlockSpec((block, d_out), lambda i: (i, 0)),
        out_shape=jax.ShapeDtypeStruct((n_out, d_out), jnp.float32),
    )(p, W := w)


def kernel(x, edge_index, W):
    n, d = x.shape
    e = edge_index.shape[1]

    k = -(-e // (NW * CH))
    e_pad = NW * k * CH
    rows_per_sub = -(-(n + 1) // (NS * CH)) * CH
    n_acc = NS * rows_per_sub

    src = edge_index[0].astype(jnp.int32)
    dst = edge_index[1].astype(jnp.int32)
    pad = e_pad - e
    if pad:
        src = jnp.concatenate([src, jnp.zeros((pad,), jnp.int32)])
        dst = jnp.concatenate([dst, jnp.full((pad,), n, jnp.int32)])
    srcm = src.reshape(NW * k, CH)
    dstm = dst.reshape(NW * k, CH)
    zeros = jnp.zeros((CH, d), jnp.float32)

    p = _sc_aggregate(x, srcm, dstm, zeros, n_acc, k, rows_per_sub)
    return _tc_combine_matmul(p, W, n, block=1000)


# speedup vs baseline: 3.1881x; 3.1881x over previous
import functools

import jax
import jax.numpy as jnp
from jax import lax
from jax.experimental import pallas as pl
from jax.experimental.pallas import tpu as pltpu
from jax.experimental.pallas import tpu_sc as plsc

NC = 2
NS = 16
NW = NC * NS
CH = 128


def _sc_aggregate(x, srcm, dstm, zeros, n_acc, k, rows_per_sub):
    d = x.shape[1]
    wchunks = rows_per_sub // CH
    mesh = plsc.VectorSubcoreMesh(core_axis_name="c", subcore_axis_name="s",
                                  num_cores=NC, num_subcores=NS)

    @functools.partial(
        pl.kernel,
        out_type=jax.ShapeDtypeStruct((NC, n_acc, d), jnp.float32),
        mesh=mesh,
        scratch_types=[
            pltpu.VMEM((k, CH), jnp.int32),
            pltpu.VMEM((k, CH), jnp.int32),
            pltpu.VMEM((CH, d), jnp.float32),
            pltpu.VMEM_SHARED((n_acc, d), jnp.float32),
            pltpu.SemaphoreType.DMA,
        ],
    )
    def body(x_hbm, src_hbm, dst_hbm, zero_hbm, p_hbm, src_v, dst_v, rows_v,
             acc, sem):
        c = lax.axis_index("c")
        s = lax.axis_index("s")
        wid = s * NC + c

        pltpu.sync_copy(zero_hbm, rows_v)
        for t in range(wchunks):
            pltpu.sync_copy(
                rows_v, acc.at[pl.ds(s * rows_per_sub + t * CH, CH)])
        plsc.subcore_barrier()

        pltpu.sync_copy(src_hbm.at[pl.ds(wid * k, k)], src_v)
        pltpu.sync_copy(dst_hbm.at[pl.ds(wid * k, k)], dst_v)

        def step(j, carry):
            pltpu.async_copy(x_hbm.at[src_v.at[j]], rows_v, sem).wait()
            pltpu.sync_copy(rows_v, acc.at[dst_v.at[j]], add=True)
            return carry

        lax.fori_loop(0, k, step, 0)
        plsc.subcore_barrier()

        def wstep(t, carry):
            r0 = s * rows_per_sub + t * CH
            pltpu.sync_copy(acc.at[pl.ds(r0, CH)], rows_v)
            pltpu.sync_copy(rows_v, p_hbm.at[c, pl.ds(r0, CH)])
            return carry

        lax.fori_loop(0, wchunks, wstep, 0)

    return body(x, srcm, dstm, zeros)


def _tc_combine_matmul(p, w, n_out, block):
    d_in, d_out = w.shape

    def body(p_ref, w_ref, o_ref):
        a = p_ref[0] + p_ref[1]
        o_ref[...] = jnp.dot(a, w_ref[...],
                             preferred_element_type=jnp.float32)

    return pl.pallas_call(
        body,
        grid=(n_out // block,),
        in_specs=[
            pl.BlockSpec((NC, block, d_in), lambda i: (0, i, 0)),
            pl.BlockSpec((d_in, d_out), lambda i: (0, 0)),
        ],
        out_specs=pl.BlockSpec((block, d_out), lambda i: (i, 0)),
        out_shape=jax.ShapeDtypeStruct((n_out, d_out), jnp.float32),
    )(p, w)


def kernel(x, edge_index, W):
    n, d = x.shape
    e = edge_index.shape[1]

    k = -(-(-(-e // (NW * CH))) // 8) * 8
    e_pad = NW * k * CH
    rows_per_sub = -(-(n + 1) // (NS * CH)) * CH
    n_acc = NS * rows_per_sub

    src = edge_index[0].astype(jnp.int32)
    dst = edge_index[1].astype(jnp.int32)
    pad = e_pad - e
    if pad:
        src = jnp.concatenate([src, jnp.zeros((pad,), jnp.int32)])
        dst = jnp.concatenate([dst, jnp.full((pad,), n, jnp.int32)])
    srcm = src.reshape(NW * k, CH)
    dstm = dst.reshape(NW * k, CH)
    zeros = jnp.zeros((CH, d), jnp.float32)

    p = _sc_aggregate(x, srcm, dstm, zeros, n_acc, k, rows_per_sub)
    return _tc_combine_matmul(p, W, n, block=1000)


# speedup vs baseline: 3.4994x; 1.0976x over previous
import functools

import jax
import jax.numpy as jnp
from jax import lax
from jax.experimental import pallas as pl
from jax.experimental.pallas import tpu as pltpu
from jax.experimental.pallas import tpu_sc as plsc

NC = 2
NS = 16
NW = NC * NS
CH = 128


def _sc_aggregate(x, srcm, dstm, zeros, n_acc, k, rows_per_sub):
    d = x.shape[1]
    wchunks = rows_per_sub // CH
    nb = 2
    H = 16
    groups = k // H
    mesh = plsc.VectorSubcoreMesh(core_axis_name="c", subcore_axis_name="s",
                                  num_cores=NC, num_subcores=NS)

    @functools.partial(
        pl.kernel,
        out_type=jax.ShapeDtypeStruct((NC, n_acc, d), jnp.float32),
        mesh=mesh,
        scratch_types=[
            pltpu.VMEM((2, H, CH), jnp.int32),
            pltpu.VMEM((2, H, CH), jnp.int32),
            pltpu.VMEM((nb, CH, d), jnp.float32),
            pltpu.VMEM_SHARED((n_acc, d), jnp.float32),
        ] + [pltpu.SemaphoreType.DMA] * (nb + 4),
    )
    def body(x_hbm, src_hbm, dst_hbm, zero_hbm, p_hbm, sidx, didx, rows_v,
             acc, *sems):
        gsem = sems[:nb]
        isem_s = sems[nb:nb + 2]
        isem_d = sems[nb + 2:nb + 4]
        c = lax.axis_index("c")
        s = lax.axis_index("s")
        wid = s * NC + c
        base = wid * k

        pltpu.sync_copy(zero_hbm, rows_v.at[0])
        for t in range(wchunks):
            pltpu.sync_copy(
                rows_v.at[0], acc.at[pl.ds(s * rows_per_sub + t * CH, CH)])

        pltpu.async_copy(src_hbm.at[pl.ds(base, H)], sidx.at[0], isem_s[0])
        pltpu.async_copy(dst_hbm.at[pl.ds(base, H)], didx.at[0], isem_d[0])
        plsc.subcore_barrier()

        def group_body(sg, ib):
            pltpu.make_async_copy(
                src_hbm.at[pl.ds(base, H)], sidx.at[ib], isem_s[ib]).wait()
            pltpu.make_async_copy(
                dst_hbm.at[pl.ds(base, H)], didx.at[ib], isem_d[ib]).wait()

            @pl.when(sg + 1 < groups)
            def _():
                nxt = base + (sg + 1) * H
                pltpu.async_copy(src_hbm.at[pl.ds(nxt, H)],
                                 sidx.at[1 - ib], isem_s[1 - ib])
                pltpu.async_copy(dst_hbm.at[pl.ds(nxt, H)],
                                 didx.at[1 - ib], isem_d[1 - ib])

            for b in range(nb):
                pltpu.async_copy(x_hbm.at[sidx.at[ib, b]], rows_v.at[b],
                                 gsem[b])
            for h in range(H):
                b = h % nb
                pltpu.make_async_copy(
                    x_hbm.at[sidx.at[ib, h]], rows_v.at[b], gsem[b]).wait()
                pltpu.sync_copy(rows_v.at[b], acc.at[didx.at[ib, h]],
                                add=True)
                if h + nb < H:
                    pltpu.async_copy(x_hbm.at[sidx.at[ib, h + nb]],
                                     rows_v.at[b], gsem[b])

        def group(sg, carry):
            @pl.when(lax.rem(sg, 2) == 0)
            def _():
                group_body(sg, 0)

            @pl.when(lax.rem(sg, 2) == 1)
            def _():
                group_body(sg, 1)

            return carry

        lax.fori_loop(0, groups, group, 0)
        plsc.subcore_barrier()

        def wstep(t, carry):
            r0 = s * rows_per_sub + t * CH
            pltpu.sync_copy(acc.at[pl.ds(r0, CH)], rows_v.at[0])
            pltpu.sync_copy(rows_v.at[0], p_hbm.at[c, pl.ds(r0, CH)])
            return carry

        lax.fori_loop(0, wchunks, wstep, 0)

    return body(x, srcm, dstm, zeros)


def _tc_combine_matmul(p, w, n_out, block):
    d_in, d_out = w.shape

    def body(p_ref, w_ref, o_ref):
        a = p_ref[0] + p_ref[1]
        o_ref[...] = jnp.dot(a, w_ref[...],
                             preferred_element_type=jnp.float32)

    return pl.pallas_call(
        body,
        grid=(n_out // block,),
        in_specs=[
            pl.BlockSpec((NC, block, d_in), lambda i: (0, i, 0)),
            pl.BlockSpec((d_in, d_out), lambda i: (0, 0)),
        ],
        out_specs=pl.BlockSpec((block, d_out), lambda i: (i, 0)),
        out_shape=jax.ShapeDtypeStruct((n_out, d_out), jnp.float32),
    )(p, w)


def kernel(x, edge_index, W):
    n, d = x.shape
    e = edge_index.shape[1]

    k = -(-(-(-e // (NW * CH))) // 16) * 16
    e_pad = NW * k * CH
    rows_per_sub = -(-(n + 1) // (NS * CH)) * CH
    n_acc = NS * rows_per_sub

    src = edge_index[0].astype(jnp.int32)
    dst = edge_index[1].astype(jnp.int32)
    pad = e_pad - e
    if pad:
        src = jnp.concatenate([src, jnp.zeros((pad,), jnp.int32)])
        dst = jnp.concatenate([dst, jnp.full((pad,), n, jnp.int32)])
    srcm = src.reshape(NW * k, CH)
    dstm = dst.reshape(NW * k, CH)
    zeros = jnp.zeros((CH, d), jnp.float32)

    p = _sc_aggregate(x, srcm, dstm, zeros, n_acc, k, rows_per_sub)
    return _tc_combine_matmul(p, W, n, block=1000)


# speedup vs baseline: 12.4693x; 3.5633x over previous
import functools

import jax
import jax.numpy as jnp
from jax import lax
from jax.experimental import pallas as pl
from jax.experimental.pallas import tpu as pltpu
from jax.experimental.pallas import tpu_sc as plsc

NC = 2
NS = 16
NW = NC * NS
CH = 128


def _sc_aggregate(x, srcm, dstm, zeros, n_acc, k, rows_per_sub):
    d = x.shape[1]
    wchunks = rows_per_sub // CH
    nb = 2
    H = 16
    groups = k // H
    mesh = plsc.VectorSubcoreMesh(core_axis_name="c", subcore_axis_name="s",
                                  num_cores=NC, num_subcores=NS)

    @functools.partial(
        pl.kernel,
        out_type=jax.ShapeDtypeStruct((NC, n_acc, d), jnp.float32),
        mesh=mesh,
        scratch_types=[
            pltpu.VMEM((2, H, CH), jnp.int32),
            pltpu.VMEM((2, H, CH), jnp.int32),
            pltpu.VMEM((nb, CH, d), jnp.float32),
            pltpu.VMEM_SHARED((n_acc, d), jnp.float32),
        ] + [pltpu.SemaphoreType.DMA] * (nb + 4),
    )
    def body(x_hbm, src_hbm, dst_hbm, zero_hbm, p_hbm, sidx, didx, rows_v,
             acc, *sems):
        gsem = sems[:nb]
        isem_s = sems[nb:nb + 2]
        isem_d = sems[nb + 2:nb + 4]
        c = lax.axis_index("c")
        s = lax.axis_index("s")
        wid = s * NC + c
        base = wid * k

        pltpu.sync_copy(zero_hbm, rows_v.at[0])
        for t in range(wchunks):
            pltpu.sync_copy(
                rows_v.at[0], acc.at[pl.ds(s * rows_per_sub + t * CH, CH)])

        pltpu.async_copy(src_hbm.at[pl.ds(base, H)], sidx.at[0], isem_s[0])
        pltpu.async_copy(dst_hbm.at[pl.ds(base, H)], didx.at[0], isem_d[0])
        plsc.subcore_barrier()

        def group_body(sg, ib):
            pltpu.make_async_copy(
                src_hbm.at[pl.ds(base, H)], sidx.at[ib], isem_s[ib]).wait()
            pltpu.make_async_copy(
                dst_hbm.at[pl.ds(base, H)], didx.at[ib], isem_d[ib]).wait()

            @pl.when(sg + 1 < groups)
            def _():
                nxt = base + (sg + 1) * H
                pltpu.async_copy(src_hbm.at[pl.ds(nxt, H)],
                                 sidx.at[1 - ib], isem_s[1 - ib])
                pltpu.async_copy(dst_hbm.at[pl.ds(nxt, H)],
                                 didx.at[1 - ib], isem_d[1 - ib])

            for b in range(nb):
                pltpu.async_copy(x_hbm.at[sidx.at[ib, b]], rows_v.at[b],
                                 gsem[b])
            for h in range(H):
                b = h % nb
                pltpu.make_async_copy(
                    x_hbm.at[sidx.at[ib, h]], rows_v.at[b], gsem[b]).wait()
                pltpu.sync_copy(rows_v.at[b], acc.at[didx.at[ib, h]],
                                add=True)
                if h + nb < H:
                    pltpu.async_copy(x_hbm.at[sidx.at[ib, h + nb]],
                                     rows_v.at[b], gsem[b])

        def group(sg, carry):
            @pl.when(lax.rem(sg, 2) == 0)
            def _():
                group_body(sg, 0)

            @pl.when(lax.rem(sg, 2) == 1)
            def _():
                group_body(sg, 1)

            return carry

        lax.fori_loop(0, groups, group, 0)
        plsc.subcore_barrier()

        def wstep(t, carry):
            r0 = s * rows_per_sub + t * CH
            pltpu.sync_copy(acc.at[pl.ds(r0, CH)], rows_v.at[0])
            pltpu.sync_copy(rows_v.at[0], p_hbm.at[c, pl.ds(r0, CH)])
            return carry

        lax.fori_loop(0, wchunks, wstep, 0)

    return body(x, srcm, dstm, zeros)


def _tc_combine_matmul(p, w, n_out, block):
    d_in, d_out = w.shape

    def body(p_ref, w_ref, o_ref):
        a = p_ref[0] + p_ref[1]
        o_ref[...] = jnp.dot(a, w_ref[...],
                             preferred_element_type=jnp.float32)

    return pl.pallas_call(
        body,
        grid=(n_out // block,),
        in_specs=[
            pl.BlockSpec((NC, block, d_in), lambda i: (0, i, 0)),
            pl.BlockSpec((d_in, d_out), lambda i: (0, 0)),
        ],
        out_specs=pl.BlockSpec((block, d_out), lambda i: (i, 0)),
        out_shape=jax.ShapeDtypeStruct((n_out, d_out), jnp.float32),
    )(p, w)


def kernel(x, edge_index, W):
    n, d = x.shape
    e = edge_index.shape[1]

    k = -(-(-(-e // (NW * CH))) // 16) * 16
    e_pad = NW * k * CH
    rows_per_sub = -(-(n + 1) // (NS * CH)) * CH
    n_acc = NS * rows_per_sub

    src = edge_index[0].astype(jnp.int32)
    dst = edge_index[1].astype(jnp.int32)
    pad = e_pad - e
    if pad:
        r = jnp.arange(pad, dtype=jnp.int32)
        src = jnp.concatenate([src, r % n])
        dst = jnp.concatenate([dst, n + r % (n_acc - n)])
    srcm = src.reshape(NW * k, CH)
    dstm = dst.reshape(NW * k, CH)
    zeros = jnp.zeros((CH, d), jnp.float32)

    p = _sc_aggregate(x, srcm, dstm, zeros, n_acc, k, rows_per_sub)
    return _tc_combine_matmul(p, W, n, block=1000)


# speedup vs baseline: 13.1879x; 1.0576x over previous
import functools

import jax
import jax.numpy as jnp
import numpy as np
from jax import lax
from jax.experimental import pallas as pl
from jax.experimental.pallas import tpu as pltpu
from jax.experimental.pallas import tpu_sc as plsc

NC = 2
NS = 16
NW = NC * NS
CH = 128
H = 16


def _sc_aggregate(x, srcm, dstm, zeros, n_acc, k, rows_per_sub):
    d = x.shape[1]
    wchunks = rows_per_sub // CH
    nb = 2
    groups = k // H
    mesh = plsc.VectorSubcoreMesh(core_axis_name="c", subcore_axis_name="s",
                                  num_cores=NC, num_subcores=NS)

    @functools.partial(
        pl.kernel,
        out_type=jax.ShapeDtypeStruct((NC, n_acc, d), jnp.float32),
        mesh=mesh,
        scratch_types=[
            pltpu.VMEM((2, H, CH), jnp.int32),
            pltpu.VMEM((2, H, CH), jnp.int32),
            pltpu.VMEM((nb, CH, d), jnp.float32),
            pltpu.VMEM_SHARED((n_acc, d), jnp.float32),
        ] + [pltpu.SemaphoreType.DMA] * (nb + 4),
    )
    def body(x_hbm, src_hbm, dst_hbm, zero_hbm, p_hbm, sidx, didx, rows_v,
             acc, *sems):
        gsem = sems[:nb]
        isem_s = sems[nb:nb + 2]
        isem_d = sems[nb + 2:nb + 4]
        c = lax.axis_index("c")
        s = lax.axis_index("s")
        wid = s * NC + c
        base = wid * k

        pltpu.sync_copy(zero_hbm, rows_v.at[0])
        for t in range(wchunks):
            pltpu.sync_copy(
                rows_v.at[0], acc.at[pl.ds(s * rows_per_sub + t * CH, CH)])

        pltpu.async_copy(src_hbm.at[pl.ds(base, H)], sidx.at[0], isem_s[0])
        pltpu.async_copy(dst_hbm.at[pl.ds(base, H)], didx.at[0], isem_d[0])
        plsc.subcore_barrier()

        def group_body(sg, ib):
            pltpu.make_async_copy(
                src_hbm.at[pl.ds(base, H)], sidx.at[ib], isem_s[ib]).wait()
            pltpu.make_async_copy(
                dst_hbm.at[pl.ds(base, H)], didx.at[ib], isem_d[ib]).wait()

            @pl.when(sg + 1 < groups)
            def _():
                nxt = base + (sg + 1) * H
                pltpu.async_copy(src_hbm.at[pl.ds(nxt, H)],
                                 sidx.at[1 - ib], isem_s[1 - ib])
                pltpu.async_copy(dst_hbm.at[pl.ds(nxt, H)],
                                 didx.at[1 - ib], isem_d[1 - ib])

            for b in range(nb):
                pltpu.async_copy(x_hbm.at[sidx.at[ib, b]], rows_v.at[b],
                                 gsem[b])
            for h in range(H):
                b = h % nb
                pltpu.make_async_copy(
                    x_hbm.at[sidx.at[ib, h]], rows_v.at[b], gsem[b]).wait()
                pltpu.sync_copy(rows_v.at[b], acc.at[didx.at[ib, h]],
                                add=True)
                if h + nb < H:
                    pltpu.async_copy(x_hbm.at[sidx.at[ib, h + nb]],
                                     rows_v.at[b], gsem[b])

        def group(sg, carry):
            @pl.when(lax.rem(sg, 2) == 0)
            def _():
                group_body(sg, 0)

            @pl.when(lax.rem(sg, 2) == 1)
            def _():
                group_body(sg, 1)

            return carry

        lax.fori_loop(0, groups, group, 0)
        plsc.subcore_barrier()

        def wstep(t, carry):
            r0 = s * rows_per_sub + t * CH
            pltpu.sync_copy(acc.at[pl.ds(r0, CH)], rows_v.at[0])
            pltpu.sync_copy(rows_v.at[0], p_hbm.at[c, pl.ds(r0, CH)])
            return carry

        lax.fori_loop(0, wchunks, wstep, 0)

    return body(x, srcm, dstm, zeros)


def _tc_combine_matmul(p, w, n_out, block):
    d_in, d_out = w.shape

    def body(p_ref, w_ref, o_ref):
        a = p_ref[0] + p_ref[1]
        o_ref[...] = jnp.dot(a, w_ref[...],
                             preferred_element_type=jnp.float32)

    return pl.pallas_call(
        body,
        grid=(n_out // block,),
        in_specs=[
            pl.BlockSpec((NC, block, d_in), lambda i: (0, i, 0)),
            pl.BlockSpec((d_in, d_out), lambda i: (0, 0)),
        ],
        out_specs=pl.BlockSpec((block, d_out), lambda i: (i, 0)),
        out_shape=jax.ShapeDtypeStruct((n_out, d_out), jnp.float32),
    )(p, w)


def kernel(x, edge_index, W):
    n, d = x.shape
    e = edge_index.shape[1]

    kq = 8 * H // np.gcd(8, H)
    k = -(-(-(-e // (NW * CH))) // kq) * kq
    e_pad = NW * k * CH
    rows_per_sub = -(-(n + 1) // (NS * CH)) * CH
    n_acc = NS * rows_per_sub

    ei = edge_index.astype(jnp.int32)
    pad = e_pad - e
    if pad:
        r = np.arange(pad, dtype=np.int32)
        pads = np.stack([r % n, n + r % (n_acc - n)])
        ei = jnp.concatenate([ei, jnp.asarray(pads)], axis=1)
    srcm = ei[0].reshape(NW * k, CH)
    dstm = ei[1].reshape(NW * k, CH)
    zeros = jnp.zeros((CH, d), jnp.float32)

    p = _sc_aggregate(x, srcm, dstm, zeros, n_acc, k, rows_per_sub)
    return _tc_combine_matmul(p, W, n, block=2000)


# speedup vs baseline: 13.9258x; 1.0559x over previous
import functools

import jax
import jax.numpy as jnp
import numpy as np
from jax import lax
from jax.experimental import pallas as pl
from jax.experimental.pallas import tpu as pltpu
from jax.experimental.pallas import tpu_sc as plsc

NC = 2
NS = 16
NW = NC * NS
CH = 128
H = 16


def _sc_aggregate(x, srcm, dstm, zeros, n_acc, k, rows_per_sub):
    d = x.shape[1]
    wchunks = rows_per_sub // CH
    nb = 2
    groups = k // H
    mesh = plsc.VectorSubcoreMesh(core_axis_name="c", subcore_axis_name="s",
                                  num_cores=NC, num_subcores=NS)

    @functools.partial(
        pl.kernel,
        out_type=jax.ShapeDtypeStruct((NC, n_acc, d), jnp.float32),
        mesh=mesh,
        scratch_types=[
            pltpu.VMEM((2, H, CH), jnp.int32),
            pltpu.VMEM((2, H, CH), jnp.int32),
            pltpu.VMEM((nb, CH, d), jnp.float32),
            pltpu.VMEM_SHARED((n_acc, d), jnp.float32),
        ] + [pltpu.SemaphoreType.DMA] * (nb + 4),
    )
    def body(x_hbm, src_hbm, dst_hbm, zero_hbm, p_hbm, sidx, didx, rows_v,
             acc, *sems):
        gsem = sems[:nb]
        isem_s = sems[nb:nb + 2]
        isem_d = sems[nb + 2:nb + 4]
        c = lax.axis_index("c")
        s = lax.axis_index("s")
        wid = s * NC + c
        base = wid * k

        pltpu.sync_copy(zero_hbm, rows_v.at[0])
        for t in range(wchunks):
            pltpu.async_copy(
                rows_v.at[0], acc.at[pl.ds(s * rows_per_sub + t * CH, CH)],
                gsem[0])

        pltpu.async_copy(src_hbm.at[pl.ds(base, H)], sidx.at[0], isem_s[0])
        pltpu.async_copy(dst_hbm.at[pl.ds(base, H)], didx.at[0], isem_d[0])
        if groups > 1:
            pltpu.async_copy(src_hbm.at[pl.ds(base + H, H)], sidx.at[1],
                             isem_s[1])
            pltpu.async_copy(dst_hbm.at[pl.ds(base + H, H)], didx.at[1],
                             isem_d[1])
        for t in range(wchunks):
            pltpu.make_async_copy(
                rows_v.at[0], acc.at[pl.ds(s * rows_per_sub, CH)],
                gsem[0]).wait()
        plsc.subcore_barrier()

        pltpu.make_async_copy(
            src_hbm.at[pl.ds(base, H)], sidx.at[0], isem_s[0]).wait()
        pltpu.make_async_copy(
            dst_hbm.at[pl.ds(base, H)], didx.at[0], isem_d[0]).wait()
        for b in range(nb):
            pltpu.async_copy(x_hbm.at[sidx.at[0, b]], rows_v.at[b], gsem[b])

        def group_body(sg, ib):
            for h in range(H):
                b = h % nb
                pltpu.make_async_copy(
                    x_hbm.at[sidx.at[ib, h]], rows_v.at[b], gsem[b]).wait()
                pltpu.sync_copy(rows_v.at[b], acc.at[didx.at[ib, h]],
                                add=True)
                if h + nb < H:
                    pltpu.async_copy(x_hbm.at[sidx.at[ib, h + nb]],
                                     rows_v.at[b], gsem[b])
                else:
                    if h == H - nb:
                        @pl.when(sg + 1 < groups)
                        def _():
                            pltpu.make_async_copy(
                                src_hbm.at[pl.ds(base, H)], sidx.at[1 - ib],
                                isem_s[1 - ib]).wait()
                            pltpu.make_async_copy(
                                dst_hbm.at[pl.ds(base, H)], didx.at[1 - ib],
                                isem_d[1 - ib]).wait()

                    @pl.when(sg + 1 < groups)
                    def _():
                        pltpu.async_copy(
                            x_hbm.at[sidx.at[1 - ib, h + nb - H]],
                            rows_v.at[b], gsem[b])

            @pl.when(sg + 2 < groups)
            def _():
                nxt = base + (sg + 2) * H
                pltpu.async_copy(src_hbm.at[pl.ds(nxt, H)], sidx.at[ib],
                                 isem_s[ib])
                pltpu.async_copy(dst_hbm.at[pl.ds(nxt, H)], didx.at[ib],
                                 isem_d[ib])

        def group(sg, carry):
            @pl.when(lax.rem(sg, 2) == 0)
            def _():
                group_body(sg, 0)

            @pl.when(lax.rem(sg, 2) == 1)
            def _():
                group_body(sg, 1)

            return carry

        lax.fori_loop(0, groups, group, 0)
        plsc.subcore_barrier()

        for t in range(wchunks):
            b = t % nb
            if t >= nb:
                pltpu.make_async_copy(
                    rows_v.at[b], p_hbm.at[c, pl.ds(s * rows_per_sub, CH)],
                    gsem[b]).wait()
            r0 = s * rows_per_sub + t * CH
            pltpu.sync_copy(acc.at[pl.ds(r0, CH)], rows_v.at[b])
            pltpu.async_copy(rows_v.at[b], p_hbm.at[c, pl.ds(r0, CH)],
                             gsem[b])
        for t in range(max(wchunks - nb, 0), wchunks):
            b = t % nb
            pltpu.make_async_copy(
                rows_v.at[b], p_hbm.at[c, pl.ds(s * rows_per_sub, CH)],
                gsem[b]).wait()

    return body(x, srcm, dstm, zeros)


def _tc_combine_matmul(p, w, n_out, block):
    d_in, d_out = w.shape

    def body(p_ref, w_ref, o_ref):
        a = p_ref[0] + p_ref[1]
        o_ref[...] = jnp.dot(a, w_ref[...],
                             preferred_element_type=jnp.float32)

    return pl.pallas_call(
        body,
        grid=(n_out // block,),
        in_specs=[
            pl.BlockSpec((NC, block, d_in), lambda i: (0, i, 0)),
            pl.BlockSpec((d_in, d_out), lambda i: (0, 0)),
        ],
        out_specs=pl.BlockSpec((block, d_out), lambda i: (i, 0)),
        out_shape=jax.ShapeDtypeStruct((n_out, d_out), jnp.float32),
    )(p, w)


def kernel(x, edge_index, W):
    n, d = x.shape
    e = edge_index.shape[1]

    kq = 8 * H // np.gcd(8, H)
    k = -(-(-(-e // (NW * CH))) // kq) * kq
    e_pad = NW * k * CH
    rows_per_sub = -(-(n + 1) // (NS * CH)) * CH
    n_acc = NS * rows_per_sub

    ei = edge_index.astype(jnp.int32)
    pad = e_pad - e
    if pad:
        r = np.arange(pad, dtype=np.int32)
        pads = np.stack([r % n, n + r % (n_acc - n)])
        ei = jnp.concatenate([ei, jnp.asarray(pads)], axis=1)
    srcm = ei[0].reshape(NW * k, CH)
    dstm = ei[1].reshape(NW * k, CH)
    zeros = jnp.zeros((CH, d), jnp.float32)

    p = _sc_aggregate(x, srcm, dstm, zeros, n_acc, k, rows_per_sub)
    return _tc_combine_matmul(p, W, n, block=2000)


# speedup vs baseline: 14.2171x; 1.0209x over previous
import functools

import jax
import jax.numpy as jnp
import numpy as np
from jax import lax
from jax.experimental import pallas as pl
from jax.experimental.pallas import tpu as pltpu
from jax.experimental.pallas import tpu_sc as plsc

NC = 2
NS = 16
NW = NC * NS
CH = 128
H = 16


def _sc_aggregate(x, eim, tail, zeros, n_acc, k, rows_per_sub, c_main):
    d = x.shape[1]
    wchunks = rows_per_sub // CH
    nb = 2
    groups = k // H
    mesh = plsc.VectorSubcoreMesh(core_axis_name="c", subcore_axis_name="s",
                                  num_cores=NC, num_subcores=NS)

    @functools.partial(
        pl.kernel,
        out_type=jax.ShapeDtypeStruct((NC, n_acc, d), jnp.float32),
        mesh=mesh,
        scratch_types=[
            pltpu.VMEM((2, H, CH), jnp.int32),
            pltpu.VMEM((2, H, CH), jnp.int32),
            pltpu.VMEM((nb, CH, d), jnp.float32),
            pltpu.VMEM_SHARED((n_acc, d), jnp.float32),
        ] + [pltpu.SemaphoreType.DMA] * (nb + 4),
    )
    def body(x_hbm, eim_hbm, tail_hbm, zero_hbm, p_hbm, sidx, didx, rows_v,
             acc, *sems):
        gsem = sems[:nb]
        isem_s = sems[nb:nb + 2]
        isem_d = sems[nb + 2:nb + 4]
        c = lax.axis_index("c")
        s = lax.axis_index("s")
        wid = s * NC + c
        base = wid * k

        def fire_idx(q0, ib):
            @pl.when(q0 < c_main)
            def _():
                pltpu.async_copy(eim_hbm.at[0, pl.ds(q0, H)], sidx.at[ib],
                                 isem_s[ib])
                pltpu.async_copy(eim_hbm.at[1, pl.ds(q0, H)], didx.at[ib],
                                 isem_d[ib])

            @pl.when(q0 >= c_main)
            def _():
                t0 = q0 - c_main
                pltpu.async_copy(tail_hbm.at[0, pl.ds(t0, H)], sidx.at[ib],
                                 isem_s[ib])
                pltpu.async_copy(tail_hbm.at[1, pl.ds(t0, H)], didx.at[ib],
                                 isem_d[ib])

        def wait_idx(ib):
            pltpu.make_async_copy(
                eim_hbm.at[0, pl.ds(0, H)], sidx.at[ib], isem_s[ib]).wait()
            pltpu.make_async_copy(
                eim_hbm.at[1, pl.ds(0, H)], didx.at[ib], isem_d[ib]).wait()

        pltpu.sync_copy(zero_hbm, rows_v.at[0])
        for t in range(wchunks):
            pltpu.async_copy(
                rows_v.at[0], acc.at[pl.ds(s * rows_per_sub + t * CH, CH)],
                gsem[0])

        fire_idx(base, 0)
        if groups > 1:
            fire_idx(base + H, 1)
        for t in range(wchunks):
            pltpu.make_async_copy(
                rows_v.at[0], acc.at[pl.ds(s * rows_per_sub, CH)],
                gsem[0]).wait()
        plsc.subcore_barrier()

        wait_idx(0)
        for b in range(nb):
            pltpu.async_copy(x_hbm.at[sidx.at[0, b]], rows_v.at[b], gsem[b])

        def group_body(sg, ib):
            for h in range(H):
                b = h % nb
                pltpu.make_async_copy(
                    x_hbm.at[sidx.at[ib, h]], rows_v.at[b], gsem[b]).wait()
                pltpu.sync_copy(rows_v.at[b], acc.at[didx.at[ib, h]],
                                add=True)
                if h + nb < H:
                    pltpu.async_copy(x_hbm.at[sidx.at[ib, h + nb]],
                                     rows_v.at[b], gsem[b])
                else:
                    if h == H - nb:
                        @pl.when(sg + 1 < groups)
                        def _():
                            wait_idx(1 - ib)

                    @pl.when(sg + 1 < groups)
                    def _():
                        pltpu.async_copy(
                            x_hbm.at[sidx.at[1 - ib, h + nb - H]],
                            rows_v.at[b], gsem[b])

            @pl.when(sg + 2 < groups)
            def _():
                fire_idx(base + (sg + 2) * H, ib)

        def group(sg, carry):
            @pl.when(lax.rem(sg, 2) == 0)
            def _():
                group_body(sg, 0)

            @pl.when(lax.rem(sg, 2) == 1)
            def _():
                group_body(sg, 1)

            return carry

        lax.fori_loop(0, groups, group, 0)
        plsc.subcore_barrier()

        for t in range(wchunks):
            b = t % nb
            if t >= nb:
                pltpu.make_async_copy(
                    rows_v.at[b], p_hbm.at[c, pl.ds(s * rows_per_sub, CH)],
                    gsem[b]).wait()
            r0 = s * rows_per_sub + t * CH
            pltpu.sync_copy(acc.at[pl.ds(r0, CH)], rows_v.at[b])
            pltpu.async_copy(rows_v.at[b], p_hbm.at[c, pl.ds(r0, CH)],
                             gsem[b])
        for t in range(max(wchunks - nb, 0), wchunks):
            b = t % nb
            pltpu.make_async_copy(
                rows_v.at[b], p_hbm.at[c, pl.ds(s * rows_per_sub, CH)],
                gsem[b]).wait()

    return body(x, eim, tail, zeros)


def _tc_combine_matmul(p, w, n_out, block):
    d_in, d_out = w.shape

    def body(p_ref, w_ref, o_ref):
        a = p_ref[0] + p_ref[1]
        o_ref[...] = jnp.dot(a, w_ref[...],
                             preferred_element_type=jnp.float32)

    return pl.pallas_call(
        body,
        grid=(n_out // block,),
        in_specs=[
            pl.BlockSpec((NC, block, d_in), lambda i: (0, i, 0)),
            pl.BlockSpec((d_in, d_out), lambda i: (0, 0)),
        ],
        out_specs=pl.BlockSpec((block, d_out), lambda i: (i, 0)),
        out_shape=jax.ShapeDtypeStruct((n_out, d_out), jnp.float32),
    )(p, w)


def kernel(x, edge_index, W):
    n, d = x.shape
    e = edge_index.shape[1]

    kq = 8 * H // np.gcd(8, H)
    k = -(-(-(-e // (NW * CH))) // kq) * kq
    e_pad = NW * k * CH
    rows_per_sub = -(-(n + 1) // (NS * CH)) * CH
    n_acc = NS * rows_per_sub

    ei = edge_index.astype(jnp.int32)
    pad = e_pad - e

    def pad_consts(npad, off):
        r = np.arange(off, off + npad, dtype=np.int32)
        return np.stack([r % n, n + r % (n_acc - n)])

    if e % CH == 0 and e // CH >= H:
        cv = e // CH
        c_main = cv // H * H
        t_chunks = NW * k - c_main
        eim = ei.reshape(2, cv, CH)
        if t_chunks:
            tail = jnp.concatenate(
                [ei[:, c_main * CH:], jnp.asarray(pad_consts(pad, 0))],
                axis=1).reshape(2, t_chunks, CH)
        else:
            tail = jnp.zeros((2, H, CH), jnp.int32)
    else:
        c_main = NW * k
        if pad:
            ei = jnp.concatenate([ei, jnp.asarray(pad_consts(pad, 0))],
                                 axis=1)
        eim = ei.reshape(2, NW * k, CH)
        tail = jnp.zeros((2, H, CH), jnp.int32)

    zeros = jnp.zeros((CH, d), jnp.float32)

    p = _sc_aggregate(x, eim, tail, zeros, n_acc, k, rows_per_sub, c_main)
    return _tc_combine_matmul(p, W, n, block=2000)


# speedup vs baseline: 14.2998x; 1.0058x over previous
import functools

import jax
import jax.numpy as jnp
import numpy as np
from jax import lax
from jax.experimental import pallas as pl
from jax.experimental.pallas import tpu as pltpu
from jax.experimental.pallas import tpu_sc as plsc

NC = 2
NS = 16
NW = NC * NS
CH = 128
H = 16


def _sc_aggregate(x, eim, tail, zeros, n_acc, k, rows_per_sub, c_main):
    d = x.shape[1]
    wchunks = rows_per_sub // CH
    nb = 2
    groups = k // H
    mesh = plsc.VectorSubcoreMesh(core_axis_name="c", subcore_axis_name="s",
                                  num_cores=NC, num_subcores=NS)

    @functools.partial(
        pl.kernel,
        out_type=jax.ShapeDtypeStruct((NC, n_acc, d), jnp.float32),
        mesh=mesh,
        scratch_types=[
            pltpu.VMEM((2, H, CH), jnp.int32),
            pltpu.VMEM((2, H, CH), jnp.int32),
            pltpu.VMEM((nb, CH, d), jnp.float32),
            pltpu.VMEM_SHARED((n_acc, d), jnp.float32),
        ] + [pltpu.SemaphoreType.DMA] * (nb + 4),
        compiler_params=pltpu.CompilerParams(use_tc_tiling_on_sc=False),
    )
    def body(x_hbm, eim_hbm, tail_hbm, zero_hbm, p_hbm, sidx, didx, rows_v,
             acc, *sems):
        gsem = sems[:nb]
        isem_s = sems[nb:nb + 2]
        isem_d = sems[nb + 2:nb + 4]
        c = lax.axis_index("c")
        s = lax.axis_index("s")
        wid = s * NC + c
        base = wid * k

        def fire_idx(q0, ib):
            @pl.when(q0 < c_main)
            def _():
                pltpu.async_copy(eim_hbm.at[0, pl.ds(q0, H)], sidx.at[ib],
                                 isem_s[ib])
                pltpu.async_copy(eim_hbm.at[1, pl.ds(q0, H)], didx.at[ib],
                                 isem_d[ib])

            @pl.when(q0 >= c_main)
            def _():
                t0 = q0 - c_main
                pltpu.async_copy(tail_hbm.at[0, pl.ds(t0, H)], sidx.at[ib],
                                 isem_s[ib])
                pltpu.async_copy(tail_hbm.at[1, pl.ds(t0, H)], didx.at[ib],
                                 isem_d[ib])

        def wait_idx(ib):
            pltpu.make_async_copy(
                eim_hbm.at[0, pl.ds(0, H)], sidx.at[ib], isem_s[ib]).wait()
            pltpu.make_async_copy(
                eim_hbm.at[1, pl.ds(0, H)], didx.at[ib], isem_d[ib]).wait()

        pltpu.sync_copy(zero_hbm, rows_v.at[0])
        for t in range(wchunks):
            pltpu.async_copy(
                rows_v.at[0], acc.at[pl.ds(s * rows_per_sub + t * CH, CH)],
                gsem[0])

        fire_idx(base, 0)
        if groups > 1:
            fire_idx(base + H, 1)
        for t in range(wchunks):
            pltpu.make_async_copy(
                rows_v.at[0], acc.at[pl.ds(s * rows_per_sub, CH)],
                gsem[0]).wait()
        plsc.subcore_barrier()

        wait_idx(0)
        for b in range(nb):
            pltpu.async_copy(x_hbm.at[sidx.at[0, b]], rows_v.at[b], gsem[b])

        def group_body(sg, ib):
            for h in range(H):
                b = h % nb
                pltpu.make_async_copy(
                    x_hbm.at[sidx.at[ib, h]], rows_v.at[b], gsem[b]).wait()
                pltpu.sync_copy(rows_v.at[b], acc.at[didx.at[ib, h]],
                                add=True)
                if h + nb < H:
                    pltpu.async_copy(x_hbm.at[sidx.at[ib, h + nb]],
                                     rows_v.at[b], gsem[b])
                else:
                    if h == H - nb:
                        @pl.when(sg + 1 < groups)
                        def _():
                            wait_idx(1 - ib)

                    @pl.when(sg + 1 < groups)
                    def _():
                        pltpu.async_copy(
                            x_hbm.at[sidx.at[1 - ib, h + nb - H]],
                            rows_v.at[b], gsem[b])

            @pl.when(sg + 2 < groups)
            def _():
                fire_idx(base + (sg + 2) * H, ib)

        def group(sg, carry):
            @pl.when(lax.rem(sg, 2) == 0)
            def _():
                group_body(sg, 0)

            @pl.when(lax.rem(sg, 2) == 1)
            def _():
                group_body(sg, 1)

            return carry

        lax.fori_loop(0, groups, group, 0)
        plsc.subcore_barrier()

        for t in range(wchunks):
            b = t % nb
            if t >= nb:
                pltpu.make_async_copy(
                    rows_v.at[b], p_hbm.at[c, pl.ds(s * rows_per_sub, CH)],
                    gsem[b]).wait()
            r0 = s * rows_per_sub + t * CH
            pltpu.sync_copy(acc.at[pl.ds(r0, CH)], rows_v.at[b])
            pltpu.async_copy(rows_v.at[b], p_hbm.at[c, pl.ds(r0, CH)],
                             gsem[b])
        for t in range(max(wchunks - nb, 0), wchunks):
            b = t % nb
            pltpu.make_async_copy(
                rows_v.at[b], p_hbm.at[c, pl.ds(s * rows_per_sub, CH)],
                gsem[b]).wait()

    return body(x, eim, tail, zeros)


def _tc_combine_matmul(p, w, n_out, block):
    d_in, d_out = w.shape

    def body(p_ref, w_ref, o_ref):
        a = p_ref[0] + p_ref[1]
        o_ref[...] = jnp.dot(a, w_ref[...],
                             preferred_element_type=jnp.float32)

    return pl.pallas_call(
        body,
        grid=(n_out // block,),
        in_specs=[
            pl.BlockSpec((NC, block, d_in), lambda i: (0, i, 0)),
            pl.BlockSpec((d_in, d_out), lambda i: (0, 0)),
        ],
        out_specs=pl.BlockSpec((block, d_out), lambda i: (i, 0)),
        out_shape=jax.ShapeDtypeStruct((n_out, d_out), jnp.float32),
    )(p, w)


def kernel(x, edge_index, W):
    n, d = x.shape
    e = edge_index.shape[1]

    kq = 8 * H // np.gcd(8, H)
    k = -(-(-(-e // (NW * CH))) // kq) * kq
    e_pad = NW * k * CH
    rows_per_sub = -(-(n + 1) // (NS * CH)) * CH
    n_acc = NS * rows_per_sub

    ei = edge_index.astype(jnp.int32)
    pad = e_pad - e

    def pad_consts(npad, off):
        r = np.arange(off, off + npad, dtype=np.int32)
        return np.stack([r % n, n + r % (n_acc - n)])

    if e % CH == 0 and e // CH >= H:
        cv = e // CH
        c_main = cv // H * H
        t_chunks = NW * k - c_main
        eim = ei.reshape(2, cv, CH)
        if t_chunks:
            tail = jnp.concatenate(
                [ei[:, c_main * CH:], jnp.asarray(pad_consts(pad, 0))],
                axis=1).reshape(2, t_chunks, CH)
        else:
            tail = jnp.zeros((2, H, CH), jnp.int32)
    else:
        c_main = NW * k
        if pad:
            ei = jnp.concatenate([ei, jnp.asarray(pad_consts(pad, 0))],
                                 axis=1)
        eim = ei.reshape(2, NW * k, CH)
        tail = jnp.zeros((2, H, CH), jnp.int32)

    zeros = jnp.zeros((CH, d), jnp.float32)

    p = _sc_aggregate(x, eim, tail, zeros, n_acc, k, rows_per_sub, c_main)
    return _tc_combine_matmul(p, W, n, block=2000)


# speedup vs baseline: 14.4564x; 1.0110x over previous
import functools

import jax
import jax.numpy as jnp
import numpy as np
from jax import lax
from jax.experimental import pallas as pl
from jax.experimental.pallas import tpu as pltpu
from jax.experimental.pallas import tpu_sc as plsc

NC = 2
NS = 16
NW = NC * NS
CH = 128
H = 16


def _sc_aggregate(x, eim, tail, zeros, n_acc, k, rows_per_sub, c_main):
    d = x.shape[1]
    wchunks = rows_per_sub // CH
    nb = 2
    groups = k // H
    mesh = plsc.VectorSubcoreMesh(core_axis_name="c", subcore_axis_name="s",
                                  num_cores=NC, num_subcores=NS)

    @functools.partial(
        pl.kernel,
        out_type=jax.ShapeDtypeStruct((NC, n_acc, d), jnp.float32),
        mesh=mesh,
        scratch_types=[
            pltpu.VMEM((2, H, CH), jnp.int32),
            pltpu.VMEM((2, H, CH), jnp.int32),
            pltpu.VMEM((nb, CH, d), jnp.float32),
            pltpu.VMEM_SHARED((n_acc, d), jnp.float32),
        ] + [pltpu.SemaphoreType.DMA] * (nb + 4),
        compiler_params=pltpu.CompilerParams(use_tc_tiling_on_sc=False),
    )
    def body(x_hbm, eim_hbm, tail_hbm, zero_hbm, p_hbm, sidx, didx, rows_v,
             acc, *sems):
        gsem = sems[:nb]
        isem_s = sems[nb:nb + 2]
        isem_d = sems[nb + 2:nb + 4]
        c = lax.axis_index("c")
        s = lax.axis_index("s")
        wid = s * NC + c
        base = wid * k

        def fire_idx(q0, ib):
            @pl.when(q0 < c_main)
            def _():
                pltpu.async_copy(eim_hbm.at[0, pl.ds(q0, H)], sidx.at[ib],
                                 isem_s[ib])
                pltpu.async_copy(eim_hbm.at[1, pl.ds(q0, H)], didx.at[ib],
                                 isem_d[ib])

            @pl.when(q0 >= c_main)
            def _():
                t0 = q0 - c_main
                pltpu.async_copy(tail_hbm.at[0, pl.ds(t0, H)], sidx.at[ib],
                                 isem_s[ib])
                pltpu.async_copy(tail_hbm.at[1, pl.ds(t0, H)], didx.at[ib],
                                 isem_d[ib])

        def wait_idx(ib):
            pltpu.make_async_copy(
                eim_hbm.at[0, pl.ds(0, H)], sidx.at[ib], isem_s[ib]).wait()
            pltpu.make_async_copy(
                eim_hbm.at[1, pl.ds(0, H)], didx.at[ib], isem_d[ib]).wait()

        pltpu.sync_copy(zero_hbm, rows_v.at[0])
        for t in range(wchunks):
            pltpu.async_copy(
                rows_v.at[0], acc.at[pl.ds(s * rows_per_sub + t * CH, CH)],
                gsem[0])

        fire_idx(base, 0)
        if groups > 1:
            fire_idx(base + H, 1)
        for t in range(wchunks):
            pltpu.make_async_copy(
                rows_v.at[0], acc.at[pl.ds(s * rows_per_sub, CH)],
                gsem[0]).wait()
        plsc.subcore_barrier()

        wait_idx(0)
        for b in range(nb):
            pltpu.async_copy(x_hbm.at[sidx.at[0, b]], rows_v.at[b], gsem[b])

        def group_body(sg, ib):
            for h in range(H):
                b = h % nb
                pltpu.make_async_copy(
                    x_hbm.at[sidx.at[ib, h]], rows_v.at[b], gsem[b]).wait()
                pltpu.sync_copy(rows_v.at[b], acc.at[didx.at[ib, h]],
                                add=True)
                if h + nb < H:
                    pltpu.async_copy(x_hbm.at[sidx.at[ib, h + nb]],
                                     rows_v.at[b], gsem[b])
                else:
                    if h == H - nb:
                        @pl.when(sg + 1 < groups)
                        def _():
                            wait_idx(1 - ib)

                    @pl.when(sg + 1 < groups)
                    def _():
                        pltpu.async_copy(
                            x_hbm.at[sidx.at[1 - ib, h + nb - H]],
                            rows_v.at[b], gsem[b])

            @pl.when(sg + 2 < groups)
            def _():
                fire_idx(base + (sg + 2) * H, ib)

        def group(sg, carry):
            @pl.when(lax.rem(sg, 2) == 0)
            def _():
                group_body(sg, 0)

            @pl.when(lax.rem(sg, 2) == 1)
            def _():
                group_body(sg, 1)

            return carry

        lax.fori_loop(0, groups, group, 0)
        plsc.subcore_barrier()

        for t in range(wchunks):
            b = t % nb
            if t >= nb:
                pltpu.make_async_copy(
                    rows_v.at[b], p_hbm.at[c, pl.ds(s * rows_per_sub, CH)],
                    gsem[b]).wait()
            r0 = s * rows_per_sub + t * CH
            pltpu.sync_copy(acc.at[pl.ds(r0, CH)], rows_v.at[b])
            pltpu.async_copy(rows_v.at[b], p_hbm.at[c, pl.ds(r0, CH)],
                             gsem[b])
        for t in range(max(wchunks - nb, 0), wchunks):
            b = t % nb
            pltpu.make_async_copy(
                rows_v.at[b], p_hbm.at[c, pl.ds(s * rows_per_sub, CH)],
                gsem[b]).wait()

    return body(x, eim, tail, zeros)


def _tc_combine_matmul(p, w, n_out, block):
    d_in, d_out = w.shape

    def body(p_ref, w_ref, o_ref):
        a = p_ref[0] + p_ref[1]
        o_ref[...] = jnp.dot(a, w_ref[...],
                             preferred_element_type=jnp.float32)

    return pl.pallas_call(
        body,
        grid=(n_out // block,),
        in_specs=[
            pl.BlockSpec((NC, block, d_in), lambda i: (0, i, 0)),
            pl.BlockSpec((d_in, d_out), lambda i: (0, 0)),
        ],
        out_specs=pl.BlockSpec((block, d_out), lambda i: (i, 0)),
        out_shape=jax.ShapeDtypeStruct((n_out, d_out), jnp.float32),
    )(p, w)


def kernel(x, edge_index, W):
    n, d = x.shape
    e = edge_index.shape[1]

    kq = 8 * H // np.gcd(8, H)
    k = -(-(-(-e // (NW * CH))) // kq) * kq
    e_pad = NW * k * CH
    rows_per_sub = -(-(n + 1) // (NS * CH)) * CH
    n_acc = NS * rows_per_sub

    ei = edge_index.astype(jnp.int32)
    pad = e_pad - e

    def pad_consts(npad, off):
        r = np.arange(off, off + npad, dtype=np.int32)
        return np.stack([r % n, n + r % (n_acc - n)])

    if e % CH == 0 and e // CH >= H:
        cv = e // CH
        c_main = cv // H * H
        t_chunks = NW * k - c_main
        eim = ei.reshape(2, cv, CH)
        if t_chunks:
            tail = jnp.concatenate(
                [ei[:, c_main * CH:], jnp.asarray(pad_consts(pad, 0))],
                axis=1).reshape(2, t_chunks, CH)
        else:
            tail = jnp.zeros((2, H, CH), jnp.int32)
    else:
        c_main = NW * k
        if pad:
            ei = jnp.concatenate([ei, jnp.asarray(pad_consts(pad, 0))],
                                 axis=1)
        eim = ei.reshape(2, NW * k, CH)
        tail = jnp.zeros((2, H, CH), jnp.int32)

    zeros = jnp.zeros((CH, d), jnp.float32)

    p = _sc_aggregate(x, eim, tail, zeros, n_acc, k, rows_per_sub, c_main)
    return _tc_combine_matmul(p, W, n, block=5000)
